# Initial kernel scaffold; baseline (speedup 1.0000x reference)
import functools

import jax
import jax.numpy as jnp
from jax import lax
from jax.experimental import pallas as pl
from jax.experimental.pallas import tpu as pltpu
from jax.experimental.pallas import tpu_sc as plsc

N = 10000
E = 320000
D = 128
NC = 2
NS = 16
NW = NC * NS
EPW = E // NW
CH = 80
NCHUNK = EPW // CH
RPT = N // NS
CW = 16


def _make_seg_kernel(with_count):
  mesh = plsc.VectorSubcoreMesh(core_axis_name="c", subcore_axis_name="s")

  out_type = [jax.ShapeDtypeStruct((NC, N, D), jnp.float32)]
  scratch = [
      pltpu.VMEM((CH,), jnp.int32),
      pltpu.VMEM((CH,), jnp.int32),
      pltpu.VMEM((CH, D), jnp.float32),
      pltpu.VMEM_SHARED((N, D), jnp.float32),
      pltpu.SemaphoreType.DMA,
  ]
  if with_count:
    out_type.append(jax.ShapeDtypeStruct((NC, N, CW), jnp.float32))
    scratch += [
        pltpu.VMEM((CH, CW), jnp.float32),
        pltpu.VMEM_SHARED((N, CW), jnp.float32),
    ]

  def body(*refs):
    if with_count:
      (feats, src, dst, zrow, zcnt, ones, out, cnt_out,
       src_v, dst_v, rows_v, acc, sem, ones_v, cacc) = refs
    else:
      (feats, src, dst, zrow, out, src_v, dst_v, rows_v, acc, sem) = refs
    c = lax.axis_index("c")
    s = lax.axis_index("s")
    wid = s * NC + c
    rbase = s * RPT
    pltpu.sync_copy(zrow.at[pl.ds(rbase, RPT)], acc.at[pl.ds(rbase, RPT)])
    if with_count:
      pltpu.sync_copy(zcnt.at[pl.ds(rbase, RPT)], cacc.at[pl.ds(rbase, RPT)])
      pltpu.sync_copy(ones, ones_v)
    plsc.subcore_barrier()

    ebase = wid * EPW

    @pl.loop(0, NCHUNK)
    def chunk(j):
      base = pl.multiple_of(ebase + j * CH, 8)
      pltpu.sync_copy(src.at[pl.ds(base, CH)], src_v)
      pltpu.sync_copy(dst.at[pl.ds(base, CH)], dst_v)
      pltpu.async_copy(feats.at[src_v], rows_v, sem).wait()
      pltpu.sync_copy(rows_v, acc.at[dst_v], add=True)
      if with_count:
        pltpu.sync_copy(ones_v, cacc.at[dst_v], add=True)

    plsc.subcore_barrier()
    pltpu.sync_copy(acc.at[pl.ds(rbase, RPT)], out.at[c, pl.ds(rbase, RPT)])
    if with_count:
      pltpu.sync_copy(cacc.at[pl.ds(rbase, RPT)],
                      cnt_out.at[c, pl.ds(rbase, RPT)])

  return pl.kernel(body, out_type=tuple(out_type), mesh=mesh,
                   scratch_types=tuple(scratch))


_seg_with_count = _make_seg_kernel(True)
_seg_plain = _make_seg_kernel(False)


def _dotT(a, w):
  return lax.dot_general(a, w, (((1,), (1,)), ((), ())),
                         preferred_element_type=jnp.float32)


def _rownorm(o):
  nrm = jnp.maximum(jnp.sqrt(jnp.sum(o * o, axis=1, keepdims=True)), 1e-12)
  return o / nrm


def _leaky(h):
  return jnp.where(h >= 0, h, 0.2 * h)


def _bn(h, g, b):
  m = jnp.mean(h, axis=0, keepdims=True)
  v = jnp.mean((h - m) ** 2, axis=0, keepdims=True)
  return (h - m) / jnp.sqrt(v + 1e-5) * g + b


def _tc1_body(x, wi, bi, g1, b1, o):
  h = _leaky(_dotT(x[...], wi[...]) + bi[...])
  o[...] = _bn(h, g1[...], b1[...])


_tc1 = pl.pallas_call(
    _tc1_body,
    out_shape=jax.ShapeDtypeStruct((N, D), jnp.float32),
)


def _tc2_body(p, cnt, h, wl, bl, wr, o):
  c = cnt[0] + cnt[1]
  cnt0 = jnp.maximum(c[:, 0:1], 1.0)
  agg = (p[0] + p[1]) / cnt0
  out = _dotT(agg, wl[...]) + bl[...] + _dotT(h[...], wr[...])
  o[...] = _leaky(_rownorm(out))


_tc2 = pl.pallas_call(
    _tc2_body,
    out_shape=jax.ShapeDtypeStruct((N, D), jnp.float32),
)


def _tc3_body(p, cnt, x1, wl, bl, wr, xs, ws, bs, g2, b2, o):
  c = cnt[0] + cnt[1]
  cnt0 = jnp.maximum(c[:, 0:1], 1.0)
  agg = (p[0] + p[1]) / cnt0
  x2 = _rownorm(_dotT(agg, wl[...]) + bl[...] + _dotT(x1[...], wr[...]))
  out = x2 + _dotT(xs[...], ws[...]) + bs[...]
  o[...] = _rownorm(_bn(out, g2[...], b2[...]))


_tc3 = pl.pallas_call(
    _tc3_body,
    out_shape=jax.ShapeDtypeStruct((N, D), jnp.float32),
)


def kernel(x, edge_index, Wi, bi, g1, b1, Wl1, bl1, Wr1, Wl2, bl2, Wr2, Ws,
           bs, g2, b2):
  ei = edge_index.astype(jnp.int32)
  src, dst = ei[0], ei[1]
  zrow = jnp.zeros((N, D), jnp.float32)
  zcnt = jnp.zeros((N, CW), jnp.float32)
  ones = jnp.ones((CH, CW), jnp.float32)
  row = lambda v: v.reshape(1, -1)

  h = _tc1(x, Wi, row(bi), row(g1), row(b1))
  p1, cnt = _seg_with_count(h, src, dst, zrow, zcnt, ones)
  x1 = _tc2(p1, cnt, h, Wl1, row(bl1), Wr1)
  (p2,) = _seg_plain(x1, src, dst, zrow)
  out = _tc3(p2, cnt, x1, Wl2, row(bl2), Wr2, h, Ws, row(bs), row(g2),
             row(b2))
  return out


# speedup vs baseline: 4.3682x; 4.3682x over previous
import functools

import jax
import jax.numpy as jnp
from jax import lax
from jax.experimental import pallas as pl
from jax.experimental.pallas import tpu as pltpu
from jax.experimental.pallas import tpu_sc as plsc

N = 10000
E = 320000
D = 128
NC = 2
NS = 16
NW = NC * NS
EPW = E // NW
CH = 80
NCHUNK = EPW // CH
NP = 10240
RPT = NP // NS
CW = 16


def _mesh():
  return plsc.VectorSubcoreMesh(core_axis_name="c", subcore_axis_name="s",
                                num_cores=NC, num_subcores=NS)


@functools.lru_cache(maxsize=None)
def _make_seg_kernel():
  scratch = (
      pltpu.VMEM((CH,), jnp.int32),
      pltpu.VMEM((CH,), jnp.int32),
      pltpu.VMEM((CH, D), jnp.float32),
      pltpu.VMEM_SHARED((NP, D), jnp.float32),
      pltpu.SemaphoreType.DMA,
  )

  def body(feats, src, dst, zrow, out, src_v, dst_v, rows_v, acc, sem):
    c = lax.axis_index("c")
    s = lax.axis_index("s")
    wid = s * NC + c
    rbase = pl.multiple_of(s * RPT, 8)
    pltpu.sync_copy(zrow.at[pl.ds(rbase, RPT)], acc.at[pl.ds(rbase, RPT)])
    plsc.subcore_barrier()

    ebase = wid * EPW

    @pl.loop(0, NCHUNK)
    def chunk(j):
      base = pl.multiple_of(ebase + j * CH, 8)
      pltpu.sync_copy(src.at[pl.ds(base, CH)], src_v)
      pltpu.sync_copy(dst.at[pl.ds(base, CH)], dst_v)
      pltpu.async_copy(feats.at[src_v], rows_v, sem).wait()
      pltpu.sync_copy(rows_v, acc.at[dst_v], add=True)

    plsc.subcore_barrier()
    pltpu.sync_copy(acc.at[pl.ds(rbase, RPT)], out.at[c, pl.ds(rbase, RPT)])

  return pl.kernel(
      body, out_type=jax.ShapeDtypeStruct((NC, NP, D), jnp.float32),
      mesh=_mesh(), scratch_types=scratch)


@functools.lru_cache(maxsize=None)
def _make_cnt_kernel():
  scratch = (
      pltpu.VMEM((CH,), jnp.int32),
      pltpu.VMEM((CH, D), jnp.float32),
      pltpu.VMEM_SHARED((NP, D), jnp.float32),
  )

  def body(dst, zrow, out, dst_v, ones_v, cacc):
    c = lax.axis_index("c")
    s = lax.axis_index("s")
    wid = s * NC + c
    rbase = pl.multiple_of(s * RPT, 8)

    @pl.loop(0, CH)
    def fill_row(i):
      @pl.loop(0, D // 16)
      def fill_lane(k):
        ones_v[i, pl.ds(k * 16, 16)] = jnp.ones((16,), jnp.float32)

    pltpu.sync_copy(zrow.at[pl.ds(rbase, RPT)], cacc.at[pl.ds(rbase, RPT)])
    plsc.subcore_barrier()

    ebase = wid * EPW

    @pl.loop(0, NCHUNK)
    def chunk(j):
      base = pl.multiple_of(ebase + j * CH, 8)
      pltpu.sync_copy(dst.at[pl.ds(base, CH)], dst_v)
      pltpu.sync_copy(ones_v, cacc.at[dst_v], add=True)

    plsc.subcore_barrier()
    pltpu.sync_copy(cacc.at[pl.ds(rbase, RPT)], out.at[c, pl.ds(rbase, RPT)])

  return pl.kernel(
      body, out_type=jax.ShapeDtypeStruct((NC, NP, D), jnp.float32),
      mesh=_mesh(), scratch_types=scratch)


def _dotT(a, w):
  return lax.dot_general(a, w, (((1,), (1,)), ((), ())),
                         preferred_element_type=jnp.float32)


def _rownorm(o):
  nrm = jnp.maximum(jnp.sqrt(jnp.sum(o * o, axis=1, keepdims=True)), 1e-12)
  return o / nrm


def _leaky(h):
  return jnp.where(h >= 0, h, 0.2 * h)


def _bn(h, g, b):
  m = jnp.mean(h, axis=0, keepdims=True)
  v = jnp.mean((h - m) ** 2, axis=0, keepdims=True)
  return (h - m) / jnp.sqrt(v + 1e-5) * g + b


def _tc1_body(x, wi, bi, g1, b1, o):
  h = _leaky(_dotT(x[...], wi[...]) + bi[...])
  o[...] = _bn(h, g1[...], b1[...])


_tc1 = pl.pallas_call(
    _tc1_body,
    out_shape=jax.ShapeDtypeStruct((N, D), jnp.float32),
)


def _tc2_body(p, cnt, h, wl, bl, wr, o):
  cnt0 = jnp.maximum(cnt[0, :N, 0:1] + cnt[1, :N, 0:1], 1.0)
  agg = (p[0, :N] + p[1, :N]) / cnt0
  out = _dotT(agg, wl[...]) + bl[...] + _dotT(h[...], wr[...])
  o[...] = _leaky(_rownorm(out))


_tc2 = pl.pallas_call(
    _tc2_body,
    out_shape=jax.ShapeDtypeStruct((N, D), jnp.float32),
)


def _tc3_body(p, cnt, x1, wl, bl, wr, xs, ws, bs, g2, b2, o):
  cnt0 = jnp.maximum(cnt[0, :N, 0:1] + cnt[1, :N, 0:1], 1.0)
  agg = (p[0, :N] + p[1, :N]) / cnt0
  x2 = _rownorm(_dotT(agg, wl[...]) + bl[...] + _dotT(x1[...], wr[...]))
  out = x2 + _dotT(xs[...], ws[...]) + bs[...]
  o[...] = _rownorm(_bn(out, g2[...], b2[...]))


_tc3 = pl.pallas_call(
    _tc3_body,
    out_shape=jax.ShapeDtypeStruct((N, D), jnp.float32),
)


def kernel(x, edge_index, Wi, bi, g1, b1, Wl1, bl1, Wr1, Wl2, bl2, Wr2, Ws,
           bs, g2, b2):
  ei = edge_index.astype(jnp.int32)
  src, dst = ei[0], ei[1]
  zrow = jnp.zeros((NP, D), jnp.float32)
  row = lambda v: v.reshape(1, -1)

  h = _tc1(x, Wi, row(bi), row(g1), row(b1))
  cnt = _make_cnt_kernel()(dst, zrow)
  p1 = _make_seg_kernel()(h, src, dst, zrow)
  x1 = _tc2(p1, cnt, h, Wl1, row(bl1), Wr1)
  p2 = _make_seg_kernel()(x1, src, dst, zrow)
  out = _tc3(p2, cnt, x1, Wl2, row(bl2), Wr2, h, Ws, row(bs), row(g2),
             row(b2))
  return out
